# baseline (speedup 1.0000x reference)
import functools

import jax
import jax.numpy as jnp
from jax.experimental import pallas as pl
from jax.experimental.pallas import tpu as pltpu

N = 32768
IN_DIM = 768
HID = 512
E = 64
TOPK = 8
EPS_LN = 1e-5

BLOCK = 4096


def _gate_kernel(x_ref, w1_ref, b1_ref, w2_ref, b2_ref,
                 idx_ref, w_ref):
    x = x_ref[...]
    mu = jnp.mean(x, axis=-1, keepdims=True)
    var = jnp.mean(jnp.square(x), axis=-1, keepdims=True) - jnp.square(mu)
    r = jax.lax.rsqrt(var + EPS_LN)
    xn = (x - mu) * r
    h = jnp.dot(xn, w1_ref[...], preferred_element_type=jnp.float32)
    h = h + b1_ref[...]
    g = h * (1.0 + jax.lax.erf(h * 0.7071067811865476))
    logits_t = jax.lax.dot_general(
        w2_ref[...], g, (((0,), (1,)), ((), ())),
        preferred_element_type=jnp.float32)
    logits_t = logits_t + b2_ref[...]
    sub = jax.lax.broadcasted_iota(jnp.int32, logits_t.shape, 0)
    work = logits_t
    idx_rows = []
    val_rows = []
    for _ in range(TOPK):
        v = jnp.max(work, axis=0, keepdims=True)
        cand = jnp.where(work == v, sub, E)
        i = jnp.min(cand, axis=0, keepdims=True)
        idx_rows.append(i)
        val_rows.append(v)
        work = jnp.where(sub == i, -jnp.inf, work)
    lsel = jnp.concatenate(val_rows, axis=0)
    idxs = jnp.concatenate(idx_rows, axis=0)
    vals = jnp.exp(lsel - lsel[0:1, :])
    denom = jnp.sum(vals, axis=0, keepdims=True) + 1e-20
    w_t = vals / denom
    idx_ref[...] = idxs.T
    w_ref[...] = w_t.T


@jax.jit
def kernel(hidden, gamma, beta, W1, b1, W2, b2):
    w1f = gamma[:, None] * W1
    b1f = (b1 + beta @ W1).reshape(1, HID)
    w2h = 0.5 * W2
    b2c = b2.reshape(E, 1)
    grid = (N // BLOCK,)
    full = lambda shape: pl.BlockSpec(shape, lambda i: (0, 0))
    out = pl.pallas_call(
        _gate_kernel,
        grid=grid,
        compiler_params=pltpu.CompilerParams(
            dimension_semantics=("parallel",)),
        in_specs=[
            pl.BlockSpec((BLOCK, IN_DIM), lambda i: (i, 0)),
            full((IN_DIM, HID)),
            full((1, HID)),
            full((HID, E)),
            full((E, 1)),
        ],
        out_specs=[
            pl.BlockSpec((BLOCK, TOPK), lambda i: (i, 0)),
            pl.BlockSpec((BLOCK, TOPK), lambda i: (i, 0)),
        ],
        out_shape=[
            jax.ShapeDtypeStruct((N, TOPK), jnp.int32),
            jax.ShapeDtypeStruct((N, TOPK), jnp.float32),
        ],
    )(hidden, w1f, b1f, w2h, b2c)
    return out[0], out[1]


# speedup vs baseline: 1.0093x; 1.0093x over previous
import jax
import jax.numpy as jnp
from jax.experimental import pallas as pl
from jax.experimental.pallas import tpu as pltpu

N = 32768
IN_DIM = 768
HID = 512
E = 64
TOPK = 8
EPS_LN = 1e-5

BLOCK = 2048


def _gate_kernel(x_ref, w1_ref, b1_ref, w2_ref, b2_ref,
                 idx_ref, w_ref):
    x = x_ref[...]
    mu = jnp.mean(x, axis=-1, keepdims=True)
    var = jnp.mean(jnp.square(x), axis=-1, keepdims=True) - jnp.square(mu)
    r = jax.lax.rsqrt(var + EPS_LN)
    xn = (x - mu) * r
    h = jnp.dot(xn, w1_ref[...], preferred_element_type=jnp.float32)
    h = h + b1_ref[...]
    g = h * (1.0 + jax.lax.erf(h * 0.7071067811865476))
    logits_t = jax.lax.dot_general(
        w2_ref[...], g, (((0,), (1,)), ((), ())),
        preferred_element_type=jnp.float32)
    logits_t = logits_t + b2_ref[...]
    sub = jax.lax.broadcasted_iota(jnp.int32, logits_t.shape, 0)
    work = logits_t
    idx_rows = []
    val_rows = []
    for k in range(TOPK):
        v = jnp.max(work, axis=0, keepdims=True)
        cand = jnp.where(work == v, sub, E)
        i = jnp.min(cand, axis=0, keepdims=True)
        idx_rows.append(i)
        val_rows.append(v)
        if k + 1 < TOPK:
            work = jnp.where(sub == i, -jnp.inf, work)
    lsel = jnp.concatenate(val_rows, axis=0)
    idxs = jnp.concatenate(idx_rows, axis=0)
    vals = jnp.exp(lsel - lsel[0:1, :])
    denom = jnp.sum(vals, axis=0, keepdims=True) + 1e-20
    w_t = vals / denom
    idx_ref[...] = idxs.T
    w_ref[...] = w_t.T


@jax.jit
def kernel(hidden, gamma, beta, W1, b1, W2, b2):
    w1f = gamma[:, None] * W1
    b1f = (b1 + beta @ W1).reshape(1, HID)
    w2h = 0.5 * W2
    b2c = b2.reshape(E, 1)
    grid = (N // BLOCK,)
    full = lambda shape: pl.BlockSpec(shape, lambda i: (0, 0))
    out = pl.pallas_call(
        _gate_kernel,
        grid=grid,
        compiler_params=pltpu.CompilerParams(
            dimension_semantics=("parallel",)),
        in_specs=[
            pl.BlockSpec((BLOCK, IN_DIM), lambda i: (i, 0)),
            full((IN_DIM, HID)),
            full((1, HID)),
            full((HID, E)),
            full((E, 1)),
        ],
        out_specs=[
            pl.BlockSpec((BLOCK, TOPK), lambda i: (i, 0)),
            pl.BlockSpec((BLOCK, TOPK), lambda i: (i, 0)),
        ],
        out_shape=[
            jax.ShapeDtypeStruct((N, TOPK), jnp.int32),
            jax.ShapeDtypeStruct((N, TOPK), jnp.float32),
        ],
    )(hidden, w1f, b1f, w2h, b2c)
    return out[0], out[1]
